# baseline (speedup 1.0000x reference)
import functools

import jax
import jax.numpy as jnp
from jax import lax
from jax.experimental import pallas as pl
from jax.experimental.pallas import tpu as pltpu
from jax.experimental.pallas import tpu_sc as plsc


def _make_gather(num_rows, B, N, D):
    info = plsc.get_sparse_core_info()
    NC, NS, L = info.num_cores, info.num_subcores, info.num_lanes
    NW = NC * NS
    assert B % NW == 0
    b_per_w = B // NW
    assert b_per_w % L == 0 and b_per_w % 8 == 0

    mesh = plsc.VectorSubcoreMesh(core_axis_name="c", subcore_axis_name="s")

    @functools.partial(
        pl.kernel,
        mesh=mesh,
        out_type=jax.ShapeDtypeStruct((B, D), jnp.float32),
        scratch_types=[
            pltpu.VMEM((b_per_w,), jnp.int32),
            pltpu.VMEM((b_per_w, D), jnp.float32),
            pltpu.SemaphoreType.DMA,
            pltpu.SemaphoreType.DMA,
            pltpu.SemaphoreType.DMA,
            pltpu.SemaphoreType.DMA,
            pltpu.SemaphoreType.DMA,
        ],
    )
    def gather(table_hbm, cur_hbm, out_hbm, idx_v, rows_v, i0, i1, g0, g1, wsem):
        wid = lax.axis_index("s") * NC + lax.axis_index("c")
        base = wid * b_per_w
        C = 2
        rpc = b_per_w // C
        isem = [i0, i1]
        gsem = [g0, g1]
        idx_cp = [
            pltpu.async_copy(
                cur_hbm.at[pl.ds(base + c * rpc, rpc)],
                idx_v.at[pl.ds(c * rpc, rpc)],
                isem[c],
            )
            for c in range(C)
        ]
        lane = lax.iota(jnp.int32, L) * N
        gathers = []
        for c in range(C):
            idx_cp[c].wait()
            for i in range(c * rpc // L, (c + 1) * rpc // L):
                sl = pl.ds(i * L, L)
                idx_v[sl] = idx_v[sl] + ((base + i * L) * N + lane)
            gathers.append(
                pltpu.async_copy(
                    table_hbm.at[idx_v.at[pl.ds(c * rpc, rpc)]],
                    rows_v.at[pl.ds(c * rpc, rpc)],
                    gsem[c],
                )
            )
        writes = []
        for c in range(C):
            gathers[c].wait()
            writes.append(
                pltpu.async_copy(
                    rows_v.at[pl.ds(c * rpc, rpc)],
                    out_hbm.at[pl.ds(base + c * rpc, rpc)],
                    wsem,
                )
            )
        for w in writes:
            w.wait()

    return gather


def kernel(nodes_or_embeddings, current_node):
    B, N, D = nodes_or_embeddings.shape
    cur = current_node
    if cur.ndim > 1:
        cur = jnp.squeeze(cur, axis=-1)
    table = nodes_or_embeddings.reshape(B * N, D)
    cur = cur.astype(jnp.int32)
    out = _make_gather(B * N, B, N, D)(table, cur)
    return out.reshape(B, 1, D)


# speedup vs baseline: 1.0086x; 1.0086x over previous
import functools

import jax
import jax.numpy as jnp
from jax import lax
from jax.experimental import pallas as pl
from jax.experimental.pallas import tpu as pltpu
from jax.experimental.pallas import tpu_sc as plsc


def _make_gather(B, N, D):
    info = plsc.get_sparse_core_info()
    NC, NS, L = info.num_cores, info.num_subcores, info.num_lanes
    NW = NC * NS
    assert B % NW == 0
    b_per_w = B // NW
    assert b_per_w % L == 0 and b_per_w % 8 == 0

    mesh = plsc.VectorSubcoreMesh(core_axis_name="c", subcore_axis_name="s")

    @functools.partial(
        pl.kernel,
        mesh=mesh,
        out_type=jax.ShapeDtypeStruct((B, D), jnp.float32),
        scratch_types=[
            pltpu.VMEM((b_per_w,), jnp.int32),
            pltpu.VMEM((b_per_w, D), jnp.float32),
            pltpu.SemaphoreType.DMA,
        ],
    )
    def gather(table_hbm, cur_hbm, out_hbm, idx_v, rows_v, sem):
        wid = lax.axis_index("s") * NC + lax.axis_index("c")
        base = wid * b_per_w
        pltpu.sync_copy(cur_hbm.at[pl.ds(base, b_per_w)], idx_v)
        lane = lax.iota(jnp.int32, L) * N
        for i in range(b_per_w // L):
            sl = pl.ds(i * L, L)
            idx_v[sl] = idx_v[sl] + ((base + i * L) * N + lane)
        pltpu.async_copy(table_hbm.at[idx_v], rows_v, sem).wait()
        pltpu.sync_copy(rows_v, out_hbm.at[pl.ds(base, b_per_w)])

    return gather


def kernel(nodes_or_embeddings, current_node):
    B, N, D = nodes_or_embeddings.shape
    cur = current_node
    if cur.ndim > 1:
        cur = jnp.squeeze(cur, axis=-1)
    table = nodes_or_embeddings.reshape(B * N, D)
    cur = cur.astype(jnp.int32)
    out = _make_gather(B, N, D)(table, cur)
    return out.reshape(B, 1, D)
